# baseline (speedup 1.0000x reference)
import functools

import jax
import jax.numpy as jnp
from jax import lax
from jax.experimental import pallas as pl
from jax.experimental.pallas import tpu as pltpu
from jax.experimental.pallas import tpu_sc as plsc


def _sc_gather(confidence, index):
    n_rows, n_cols = confidence.shape
    b = index.shape[0]
    info = plsc.get_sparse_core_info()
    nw = info.num_cores * info.num_subcores
    b_per_w = b // nw
    mesh = plsc.VectorSubcoreMesh(core_axis_name="c", subcore_axis_name="s")

    @functools.partial(
        pl.kernel,
        mesh=mesh,
        out_type=jax.ShapeDtypeStruct((b, n_cols), jnp.float32),
        scratch_types=[
            pltpu.VMEM((b_per_w,), jnp.int32),
            pltpu.VMEM((b_per_w, n_cols), jnp.float32),
            pltpu.SemaphoreType.DMA,
        ],
        compiler_params=pltpu.CompilerParams(use_tc_tiling_on_sc=False),
    )
    def gather_kernel(table_hbm, idx_hbm, out_hbm, idx_v, rows_v, sem):
        wid = lax.axis_index("s") * info.num_cores + lax.axis_index("c")
        base = wid * b_per_w
        pltpu.sync_copy(idx_hbm.at[pl.ds(base, b_per_w)], idx_v)
        pltpu.async_copy(table_hbm.at[idx_v], rows_v, sem).wait()
        pltpu.sync_copy(rows_v, out_hbm.at[pl.ds(base, b_per_w)])

    return gather_kernel(confidence, index)


def _loss_body(ow_ref, os_ref, tg_ref, acc_ref):
    i = pl.program_id(0)
    ow = ow_ref[...]
    osl = os_ref[...]
    t = tg_ref[...]

    mw = jnp.max(ow, axis=1, keepdims=True)
    ew = jnp.exp(ow - mw)
    sw = jnp.sum(ew, axis=1, keepdims=True)
    lse_w = mw + jnp.log(sw)

    ms = jnp.max(osl, axis=1, keepdims=True)
    es = jnp.exp(osl - ms)
    ss = jnp.sum(es, axis=1, keepdims=True)
    lse_s = ms + jnp.log(ss)

    pos = t > 0.0
    safe_t = jnp.where(pos, t, 1.0)
    xlogy = t * jnp.log(safe_t)
    tsum = jnp.sum(t, axis=1, keepdims=True)

    part = (2.0 * jnp.sum(xlogy)
            - jnp.sum(t * ow) - jnp.sum(t * osl)
            + jnp.sum(tsum * (lse_w + lse_s)))

    @pl.when(i == 0)
    def _():
        acc_ref[0, 0] = 0.0

    acc_ref[0, 0] += part

    any_zero = jnp.sum((t == 0.0).astype(jnp.float32)) > 0.0

    @pl.when(any_zero)
    def _():
        pred_w = ew / sw
        pred_s = es / ss
        neg = (t == 0.0).astype(jnp.float32)
        sup = neg * (-jnp.log(jnp.abs(1.0 - pred_w) + 1e-9)
                     - jnp.log(jnp.abs(1.0 - pred_s) + 1e-9))
        acc_ref[0, 0] += jnp.sum(sup)


def _loss_tc(output_w, output_s, target, block_rows=256, interpret=False):
    b, c = output_w.shape
    grid = b // block_rows
    acc = pl.pallas_call(
        _loss_body,
        grid=(grid,),
        in_specs=[
            pl.BlockSpec((block_rows, c), lambda i: (i, 0)),
            pl.BlockSpec((block_rows, c), lambda i: (i, 0)),
            pl.BlockSpec((block_rows, c), lambda i: (i, 0)),
        ],
        out_specs=pl.BlockSpec((1, 1), lambda i: (0, 0),
                               memory_space=pltpu.SMEM),
        out_shape=jax.ShapeDtypeStruct((1, 1), jnp.float32),
        interpret=interpret,
    )(output_w, output_s, target)
    return acc[0, 0] / b


def kernel(output_w, output_s, feat_w, feat_s, confidence, index):
    del feat_w, feat_s
    target = _sc_gather(confidence, index)
    return _loss_tc(output_w, output_s, target)


# speedup vs baseline: 4.3091x; 4.3091x over previous
import functools

import jax
import jax.numpy as jnp
from jax import lax
from jax.experimental import pallas as pl
from jax.experimental.pallas import tpu as pltpu
from jax.experimental.pallas import tpu_sc as plsc


def _sc_gather(confidence, index):
    n_rows, n_cols = confidence.shape
    b = index.shape[0]
    info = plsc.get_sparse_core_info()
    nw = info.num_cores * info.num_subcores
    b_per_w = b // nw
    mesh = plsc.VectorSubcoreMesh(core_axis_name="c", subcore_axis_name="s")

    @functools.partial(
        pl.kernel,
        mesh=mesh,
        out_type=jax.ShapeDtypeStruct((b, n_cols), jnp.float32),
        scratch_types=[
            pltpu.VMEM((b_per_w,), jnp.int32),
            pltpu.VMEM((b_per_w, n_cols), jnp.float32),
            pltpu.SemaphoreType.DMA,
        ],
        compiler_params=pltpu.CompilerParams(use_tc_tiling_on_sc=False),
    )
    def gather_kernel(table_hbm, idx_hbm, out_hbm, idx_v, rows_v, sem):
        wid = lax.axis_index("s") * info.num_cores + lax.axis_index("c")
        base = wid * b_per_w
        pltpu.sync_copy(idx_hbm.at[pl.ds(base, b_per_w)], idx_v)
        pltpu.async_copy(table_hbm.at[idx_v], rows_v, sem).wait()
        pltpu.sync_copy(rows_v, out_hbm.at[pl.ds(base, b_per_w)])

    return gather_kernel(confidence, index)


def _loss_body(ow_ref, os_ref, tg_ref, acc_ref):
    i = pl.program_id(0)
    ow = ow_ref[...]
    osl = os_ref[...]
    t = tg_ref[...]

    mw = jnp.max(ow, axis=1, keepdims=True)
    ew = jnp.exp(ow - mw)
    sw = jnp.sum(ew, axis=1, keepdims=True)
    lse_w = mw + jnp.log(sw)

    ms = jnp.max(osl, axis=1, keepdims=True)
    es = jnp.exp(osl - ms)
    ss = jnp.sum(es, axis=1, keepdims=True)
    lse_s = ms + jnp.log(ss)

    pos = t > 0.0
    safe_t = jnp.where(pos, t, 1.0)
    xlogy = t * jnp.log(safe_t)
    tsum = jnp.sum(t, axis=1, keepdims=True)

    part = (2.0 * jnp.sum(xlogy)
            - jnp.sum(t * ow) - jnp.sum(t * osl)
            + jnp.sum(tsum * (lse_w + lse_s)))

    @pl.when(i == 0)
    def _():
        acc_ref[0, 0] = 0.0

    acc_ref[0, 0] += part

    any_zero = jnp.sum((t == 0.0).astype(jnp.float32)) > 0.0

    @pl.when(any_zero)
    def _():
        pred_w = ew / sw
        pred_s = es / ss
        neg = (t == 0.0).astype(jnp.float32)
        sup = neg * (-jnp.log(jnp.abs(1.0 - pred_w) + 1e-9)
                     - jnp.log(jnp.abs(1.0 - pred_s) + 1e-9))
        acc_ref[0, 0] += jnp.sum(sup)


def _loss_tc(output_w, output_s, target, block_rows=256, interpret=False):
    b, c = output_w.shape
    grid = b // block_rows
    acc = pl.pallas_call(
        _loss_body,
        grid=(grid,),
        in_specs=[
            pl.BlockSpec((block_rows, c), lambda i: (i, 0)),
            pl.BlockSpec((block_rows, c), lambda i: (i, 0)),
            pl.BlockSpec((block_rows, c), lambda i: (i, 0)),
        ],
        out_specs=pl.BlockSpec((1, 1), lambda i: (0, 0),
                               memory_space=pltpu.SMEM),
        out_shape=jax.ShapeDtypeStruct((1, 1), jnp.float32),
        interpret=interpret,
    )(output_w, output_s, target)
    return acc[0, 0] / b


def _fused_body(idx_ref, ow_ref, os_ref, conf_ref, acc_ref, tgt, sem):
    i = pl.program_id(0)
    ngrid = pl.num_programs(0)
    block_rows = ow_ref.shape[0]
    c = ow_ref.shape[1]

    def issue(step, slot):
        base = step * block_rows

        def one(j, _):
            r = idx_ref[base + j]
            pltpu.make_async_copy(
                conf_ref.at[pl.ds(r, 1), :],
                tgt.at[slot, pl.ds(j, 1), :],
                sem.at[slot],
            ).start()
            return 0

        lax.fori_loop(0, block_rows, one, 0)

    @pl.when(i == 0)
    def _():
        issue(0, 0)

    @pl.when(i + 1 < ngrid)
    def _():
        issue(i + 1, (i + 1) % 2)

    slot = i % 2
    pltpu.make_async_copy(
        conf_ref.at[pl.ds(0, block_rows), :], tgt.at[slot], sem.at[slot]
    ).wait()

    ow = ow_ref[...]
    osl = os_ref[...]
    t = tgt[slot]

    mw = jnp.max(ow, axis=1, keepdims=True)
    ew = jnp.exp(ow - mw)
    sw = jnp.sum(ew, axis=1, keepdims=True)
    lse_w = mw + jnp.log(sw)

    ms = jnp.max(osl, axis=1, keepdims=True)
    es = jnp.exp(osl - ms)
    ss = jnp.sum(es, axis=1, keepdims=True)
    lse_s = ms + jnp.log(ss)

    pos = t > 0.0
    safe_t = jnp.where(pos, t, 1.0)
    xlogy = t * jnp.log(safe_t)
    tsum = jnp.sum(t, axis=1, keepdims=True)

    part = (2.0 * jnp.sum(xlogy)
            - jnp.sum(t * ow) - jnp.sum(t * osl)
            + jnp.sum(tsum * (lse_w + lse_s)))

    @pl.when(i == 0)
    def _():
        acc_ref[0, 0] = 0.0

    acc_ref[0, 0] += part

    any_zero = jnp.sum((t == 0.0).astype(jnp.float32)) > 0.0

    @pl.when(any_zero)
    def _():
        pred_w = ew / sw
        pred_s = es / ss
        neg = (t == 0.0).astype(jnp.float32)
        sup = neg * (-jnp.log(jnp.abs(1.0 - pred_w) + 1e-9)
                     - jnp.log(jnp.abs(1.0 - pred_s) + 1e-9))
        acc_ref[0, 0] += jnp.sum(sup)


def _fused_tc(output_w, output_s, confidence, index, block_rows=256):
    b, c = output_w.shape
    grid = b // block_rows
    grid_spec = pltpu.PrefetchScalarGridSpec(
        num_scalar_prefetch=1,
        grid=(grid,),
        in_specs=[
            pl.BlockSpec((block_rows, c), lambda i, idx: (i, 0)),
            pl.BlockSpec((block_rows, c), lambda i, idx: (i, 0)),
            pl.BlockSpec(memory_space=pl.ANY),
        ],
        out_specs=pl.BlockSpec((1, 1), lambda i, idx: (0, 0),
                               memory_space=pltpu.SMEM),
        scratch_shapes=[
            pltpu.VMEM((2, block_rows, c), jnp.float32),
            pltpu.SemaphoreType.DMA((2,)),
        ],
    )
    acc = pl.pallas_call(
        _fused_body,
        grid_spec=grid_spec,
        out_shape=jax.ShapeDtypeStruct((1, 1), jnp.float32),
    )(index, output_w, output_s, confidence)
    return acc[0, 0] / b


def kernel(output_w, output_s, feat_w, feat_s, confidence, index):
    del feat_w, feat_s
    return _fused_tc(output_w, output_s, confidence, index)


# speedup vs baseline: 4.5520x; 1.0564x over previous
import functools

import jax
import jax.numpy as jnp
from jax import lax
from jax.experimental import pallas as pl
from jax.experimental.pallas import tpu as pltpu
from jax.experimental.pallas import tpu_sc as plsc


def _sc_gather(confidence, index):
    n_rows, n_cols = confidence.shape
    b = index.shape[0]
    info = plsc.get_sparse_core_info()
    nw = info.num_cores * info.num_subcores
    b_per_w = b // nw
    mesh = plsc.VectorSubcoreMesh(core_axis_name="c", subcore_axis_name="s")

    @functools.partial(
        pl.kernel,
        mesh=mesh,
        out_type=jax.ShapeDtypeStruct((b, n_cols), jnp.float32),
        scratch_types=[
            pltpu.VMEM((b_per_w,), jnp.int32),
            pltpu.VMEM((b_per_w, n_cols), jnp.float32),
            pltpu.SemaphoreType.DMA,
        ],
        compiler_params=pltpu.CompilerParams(use_tc_tiling_on_sc=False),
    )
    def gather_kernel(table_hbm, idx_hbm, out_hbm, idx_v, rows_v, sem):
        wid = lax.axis_index("s") * info.num_cores + lax.axis_index("c")
        base = wid * b_per_w
        pltpu.sync_copy(idx_hbm.at[pl.ds(base, b_per_w)], idx_v)
        pltpu.async_copy(table_hbm.at[idx_v], rows_v, sem).wait()
        pltpu.sync_copy(rows_v, out_hbm.at[pl.ds(base, b_per_w)])

    return gather_kernel(confidence, index)


def _loss_body(ow_ref, os_ref, tg_ref, acc_ref):
    i = pl.program_id(0)
    ow = ow_ref[...]
    osl = os_ref[...]
    t = tg_ref[...]

    mw = jnp.max(ow, axis=1, keepdims=True)
    ew = jnp.exp(ow - mw)
    sw = jnp.sum(ew, axis=1, keepdims=True)
    lse_w = mw + jnp.log(sw)

    ms = jnp.max(osl, axis=1, keepdims=True)
    es = jnp.exp(osl - ms)
    ss = jnp.sum(es, axis=1, keepdims=True)
    lse_s = ms + jnp.log(ss)

    pos = t > 0.0
    safe_t = jnp.where(pos, t, 1.0)
    xlogy = t * jnp.log(safe_t)
    tsum = jnp.sum(t, axis=1, keepdims=True)

    part = (2.0 * jnp.sum(xlogy)
            - jnp.sum(t * ow) - jnp.sum(t * osl)
            + jnp.sum(tsum * (lse_w + lse_s)))

    @pl.when(i == 0)
    def _():
        acc_ref[0, 0] = 0.0

    acc_ref[0, 0] += part

    any_zero = jnp.sum((t == 0.0).astype(jnp.float32)) > 0.0

    @pl.when(any_zero)
    def _():
        pred_w = ew / sw
        pred_s = es / ss
        neg = (t == 0.0).astype(jnp.float32)
        sup = neg * (-jnp.log(jnp.abs(1.0 - pred_w) + 1e-9)
                     - jnp.log(jnp.abs(1.0 - pred_s) + 1e-9))
        acc_ref[0, 0] += jnp.sum(sup)


def _loss_tc(output_w, output_s, target, block_rows=256, interpret=False):
    b, c = output_w.shape
    grid = b // block_rows
    acc = pl.pallas_call(
        _loss_body,
        grid=(grid,),
        in_specs=[
            pl.BlockSpec((block_rows, c), lambda i: (i, 0)),
            pl.BlockSpec((block_rows, c), lambda i: (i, 0)),
            pl.BlockSpec((block_rows, c), lambda i: (i, 0)),
        ],
        out_specs=pl.BlockSpec((1, 1), lambda i: (0, 0),
                               memory_space=pltpu.SMEM),
        out_shape=jax.ShapeDtypeStruct((1, 1), jnp.float32),
        interpret=interpret,
    )(output_w, output_s, target)
    return acc[0, 0] / b


_NQ = 8


def _fused_body(idx_ref, ow_ref, os_ref, conf_ref, acc_ref, tgt, sem):
    i = pl.program_id(0)
    ngrid = pl.num_programs(0)
    block_rows = ow_ref.shape[0]
    c = ow_ref.shape[1]

    def issue(step, slot):
        base = step * block_rows

        def one(j, _):
            jj = j * _NQ
            for k in range(_NQ):
                r = idx_ref[base + jj + k]
                pltpu.make_async_copy(
                    conf_ref.at[pl.ds(r, 1), :],
                    tgt.at[slot, pl.ds(jj + k, 1), :],
                    sem.at[slot, k],
                ).start()
            return 0

        lax.fori_loop(0, block_rows // _NQ, one, 0)

    @pl.when(i == 0)
    def _():
        issue(0, 0)

    @pl.when(i + 1 < ngrid)
    def _():
        issue(i + 1, (i + 1) % 2)

    slot = i % 2
    for k in range(_NQ):
        pltpu.make_async_copy(
            conf_ref.at[pl.ds(0, block_rows // _NQ), :],
            tgt.at[slot, pl.ds(0, block_rows // _NQ), :],
            sem.at[slot, k],
        ).wait()

    ow = ow_ref[...]
    osl = os_ref[...]
    t = tgt[slot]

    mw = jnp.max(ow, axis=1, keepdims=True)
    ew = jnp.exp(ow - mw)
    sw = jnp.sum(ew, axis=1, keepdims=True)
    lse_w = mw + jnp.log(sw)

    ms = jnp.max(osl, axis=1, keepdims=True)
    es = jnp.exp(osl - ms)
    ss = jnp.sum(es, axis=1, keepdims=True)
    lse_s = ms + jnp.log(ss)

    pos = t > 0.0
    safe_t = jnp.where(pos, t, 1.0)
    xlogy = t * jnp.log(safe_t)
    tsum = jnp.sum(t, axis=1, keepdims=True)

    part = (2.0 * jnp.sum(xlogy)
            - jnp.sum(t * ow) - jnp.sum(t * osl)
            + jnp.sum(tsum * (lse_w + lse_s)))

    @pl.when(i == 0)
    def _():
        acc_ref[0, 0] = 0.0

    acc_ref[0, 0] += part

    any_zero = jnp.sum((t == 0.0).astype(jnp.float32)) > 0.0

    @pl.when(any_zero)
    def _():
        pred_w = ew / sw
        pred_s = es / ss
        neg = (t == 0.0).astype(jnp.float32)
        sup = neg * (-jnp.log(jnp.abs(1.0 - pred_w) + 1e-9)
                     - jnp.log(jnp.abs(1.0 - pred_s) + 1e-9))
        acc_ref[0, 0] += jnp.sum(sup)


def _fused_tc(output_w, output_s, confidence, index, block_rows=256):
    b, c = output_w.shape
    grid = b // block_rows
    grid_spec = pltpu.PrefetchScalarGridSpec(
        num_scalar_prefetch=1,
        grid=(grid,),
        in_specs=[
            pl.BlockSpec((block_rows, c), lambda i, idx: (i, 0)),
            pl.BlockSpec((block_rows, c), lambda i, idx: (i, 0)),
            pl.BlockSpec(memory_space=pl.ANY),
        ],
        out_specs=pl.BlockSpec((1, 1), lambda i, idx: (0, 0),
                               memory_space=pltpu.SMEM),
        scratch_shapes=[
            pltpu.VMEM((2, block_rows, c), jnp.float32),
            pltpu.SemaphoreType.DMA((2, _NQ)),
        ],
    )
    acc = pl.pallas_call(
        _fused_body,
        grid_spec=grid_spec,
        out_shape=jax.ShapeDtypeStruct((1, 1), jnp.float32),
    )(index, output_w, output_s, confidence)
    return acc[0, 0] / b


def kernel(output_w, output_s, feat_w, feat_s, confidence, index):
    del feat_w, feat_s
    return _fused_tc(output_w, output_s, confidence, index)
